# baseline (speedup 1.0000x reference)
import functools

import jax
import jax.numpy as jnp
from jax import lax
from jax.experimental import pallas as pl
from jax.experimental.pallas import tpu as pltpu
from jax.experimental.pallas import tpu_sc as plsc


def kernel(inputs, pos_embedding):
    B, S = inputs.shape
    D = pos_embedding.shape[1]

    mesh = plsc.VectorSubcoreMesh(core_axis_name="c", subcore_axis_name="s")
    NC, NS = mesh.num_cores, mesh.num_subcores
    NW = NC * NS
    rows_w = S // NW

    @functools.partial(
        pl.kernel,
        mesh=mesh,
        out_type=jax.ShapeDtypeStruct((B * S, D), jnp.float32),
        scratch_types=[
            pltpu.SemaphoreType.DMA,
        ],
    )
    def sc_broadcast(table_hbm, out_hbm, sem):
        wid = lax.axis_index("s") * NC + lax.axis_index("c")
        base = wid * rows_w
        cps = [
            pltpu.async_copy(
                table_hbm.at[pl.ds(base, rows_w)],
                out_hbm.at[pl.ds(b * S + base, rows_w)],
                sem)
            for b in range(B)
        ]
        for cp in cps:
            cp.wait()

    return sc_broadcast(pos_embedding).reshape(B, S, D)


# speedup vs baseline: 55.4570x; 55.4570x over previous
import functools

import jax
import jax.numpy as jnp
from jax import lax
from jax.experimental import pallas as pl
from jax.experimental.pallas import tpu as pltpu
from jax.experimental.pallas import tpu_sc as plsc


def kernel(inputs, pos_embedding):
    B, S = inputs.shape
    D = pos_embedding.shape[1]

    mesh = plsc.VectorSubcoreMesh(core_axis_name="c", subcore_axis_name="s")
    NC, NS = mesh.num_cores, mesh.num_subcores
    NW = NC * NS
    rows_w = S // NW
    R = min(rows_w, 64)
    n_chunks = rows_w // R

    @functools.partial(
        pl.kernel,
        mesh=mesh,
        out_type=jax.ShapeDtypeStruct((B * S, D), jnp.float32),
        scratch_types=[
            pltpu.VMEM((R, D), jnp.float32),
            pltpu.SemaphoreType.DMA,
        ],
    )
    def sc_broadcast(table_hbm, out_hbm, buf, sem):
        wid = lax.axis_index("s") * NC + lax.axis_index("c")
        base = wid * rows_w
        for c in range(n_chunks):
            off = base + c * R
            pltpu.async_copy(table_hbm.at[pl.ds(off, R)], buf, sem).wait()
            for b in range(B):
                pltpu.sync_copy(buf, out_hbm.at[pl.ds(b * S + off, R)])

    return sc_broadcast(pos_embedding).reshape(B, S, D)
